# baseline (speedup 1.0000x reference)
import jax
import jax.numpy as jnp
from jax.experimental import pallas as pl
from jax.experimental.pallas import tpu as pltpu

NUM_CLASSES = 1000
ROWS = 4096
COLS = 26
FLAT = ROWS * COLS
BLOCK = 512


def _onehot_block(x_ref, o_ref):
    idx = x_ref[...]
    iota = jax.lax.broadcasted_iota(jnp.int32, (BLOCK, NUM_CLASSES), 1)
    o_ref[...] = (iota == idx).astype(jnp.float32)


def kernel(x):
    xf = x.astype(jnp.int32).reshape(FLAT, 1)
    out = pl.pallas_call(
        _onehot_block,
        grid=(FLAT // BLOCK,),
        in_specs=[pl.BlockSpec((BLOCK, 1), lambda i: (i, 0))],
        out_specs=pl.BlockSpec((BLOCK, NUM_CLASSES), lambda i: (i, 0)),
        out_shape=jax.ShapeDtypeStruct((FLAT, NUM_CLASSES), jnp.float32),
    )(xf)
    return out.reshape(ROWS, COLS, NUM_CLASSES)
